# baseline (speedup 1.0000x reference)
import jax
import jax.numpy as jnp
from jax.experimental import pallas as pl
from jax.experimental.pallas import tpu as pltpu


_BM = 400


def _xw_kernel(x_ref, w_ref, xw_ref):
    xw = jnp.dot(x_ref[...].astype(jnp.bfloat16),
                 w_ref[...].astype(jnp.bfloat16),
                 preferred_element_type=jnp.float32)
    xw_ref[...] = xw.astype(jnp.bfloat16)


def _gcn_kernel(a_ref, xw_ref, x_ref, o_ref):
    acc = jnp.dot(a_ref[...].astype(jnp.bfloat16), xw_ref[...],
                  preferred_element_type=jnp.float32)
    mask = jnp.any(x_ref[...] != 0, axis=-1, keepdims=True)
    o_ref[...] = jnp.where(mask, jnp.maximum(acc, 0.0), 0.0)


def kernel(x, a, kernel):
    n, d_in = x.shape[1], x.shape[2]
    d_out = kernel.shape[1]
    x2 = x[0]
    a2 = a[0]

    xw = pl.pallas_call(
        _xw_kernel,
        out_shape=jax.ShapeDtypeStruct((n, d_out), jnp.bfloat16),
    )(x2, kernel)

    grid = (n // _BM,)
    out = pl.pallas_call(
        _gcn_kernel,
        grid=grid,
        in_specs=[
            pl.BlockSpec((_BM, n), lambda i: (i, 0)),
            pl.BlockSpec((n, d_out), lambda i: (0, 0)),
            pl.BlockSpec((_BM, d_in), lambda i: (i, 0)),
        ],
        out_specs=pl.BlockSpec((_BM, d_out), lambda i: (i, 0)),
        out_shape=jax.ShapeDtypeStruct((n, d_out), jnp.float32),
        compiler_params=pltpu.CompilerParams(
            dimension_semantics=(pltpu.PARALLEL,),
        ),
    )(a2, xw, x2)

    return out[None]


# speedup vs baseline: 1.0452x; 1.0452x over previous
import functools

import jax
import jax.numpy as jnp
from jax.experimental import pallas as pl
from jax.experimental.pallas import tpu as pltpu


_CH = 200


def _outer_kernel(a_hbm, x_ref, w_ref, o_hbm, xw_ref, *, n, ch, d_out):
    xw = jnp.dot(x_ref[...].astype(jnp.bfloat16),
                 w_ref[...].astype(jnp.bfloat16),
                 preferred_element_type=jnp.float32)
    xw_ref[...] = xw.astype(jnp.bfloat16)

    def inner(idx, a_chunk, o_chunk):
        i = idx[0]
        acc = jnp.dot(a_chunk[...].astype(jnp.bfloat16), xw_ref[...],
                      preferred_element_type=jnp.float32)
        x_blk = x_ref[pl.ds(i * ch, ch), :]
        mask = jnp.any(x_blk != 0, axis=-1, keepdims=True)
        o_chunk[...] = jnp.where(mask, jnp.maximum(acc, 0.0), 0.0)

    pipeline = pltpu.emit_pipeline(
        inner,
        grid=(n // ch,),
        in_specs=[
            pl.BlockSpec((ch, n), lambda i: (i, 0),
                         pipeline_mode=pl.Buffered(buffer_count=4)),
        ],
        out_specs=[pl.BlockSpec((ch, d_out), lambda i: (i, 0))],
        _explicit_indices=True,
    )
    pipeline(a_hbm, o_hbm)


def kernel(x, a, kernel):
    n, d_in = x.shape[1], x.shape[2]
    d_out = kernel.shape[1]
    x2 = x[0]
    a2 = a[0]

    out = pl.pallas_call(
        functools.partial(_outer_kernel, n=n, ch=_CH, d_out=d_out),
        in_specs=[
            pl.BlockSpec(memory_space=pltpu.MemorySpace.HBM),
            pl.BlockSpec(memory_space=pltpu.MemorySpace.VMEM),
            pl.BlockSpec(memory_space=pltpu.MemorySpace.VMEM),
        ],
        out_specs=pl.BlockSpec(memory_space=pltpu.MemorySpace.HBM),
        out_shape=jax.ShapeDtypeStruct((n, d_out), jnp.float32),
        scratch_shapes=[pltpu.VMEM((n, d_out), jnp.bfloat16)],
    )(a2, x2, kernel)

    return out[None]


# speedup vs baseline: 1.0745x; 1.0280x over previous
import functools

import jax
import jax.numpy as jnp
from jax.experimental import pallas as pl
from jax.experimental.pallas import tpu as pltpu


_BM = 400


def _gcn_kernel(a_ref, x_ref, w_ref, o_ref, xw_ref, *, bm):
    i = pl.program_id(0)

    @pl.when(i == 0)
    def _():
        xw = jnp.dot(x_ref[...].astype(jnp.bfloat16),
                     w_ref[...].astype(jnp.bfloat16),
                     preferred_element_type=jnp.float32)
        xw_ref[...] = xw.astype(jnp.bfloat16)

    acc = jnp.dot(a_ref[...].astype(jnp.bfloat16), xw_ref[...],
                  preferred_element_type=jnp.float32)
    x_blk = x_ref[pl.ds(i * bm, bm), :]
    mask = jnp.any(x_blk != 0, axis=-1, keepdims=True)
    o_ref[...] = jnp.where(mask, jnp.maximum(acc, 0.0), 0.0)


def kernel(x, a, kernel):
    n, d_in = x.shape[1], x.shape[2]
    d_out = kernel.shape[1]
    x2 = x[0]
    a2 = a[0]

    grid = (n // _BM,)
    out = pl.pallas_call(
        functools.partial(_gcn_kernel, bm=_BM),
        grid=grid,
        in_specs=[
            pl.BlockSpec((_BM, n), lambda i: (i, 0)),
            pl.BlockSpec((n, d_in), lambda i: (0, 0)),
            pl.BlockSpec((d_in, d_out), lambda i: (0, 0)),
        ],
        out_specs=pl.BlockSpec((_BM, d_out), lambda i: (i, 0)),
        out_shape=jax.ShapeDtypeStruct((n, d_out), jnp.float32),
        scratch_shapes=[pltpu.VMEM((n, d_out), jnp.bfloat16)],
    )(a2, x2, kernel)

    return out[None]
